# baseline (speedup 1.0000x reference)
import functools

import jax
import jax.numpy as jnp
from jax import lax
from jax.experimental import pallas as pl
from jax.experimental.pallas import tpu as pltpu
from jax.experimental.pallas import tpu_sc as plsc

N_DET = 16
N_SO = 512
N_E = 32
BATCH = 4096
LANES = 128

_info = plsc.get_sparse_core_info()
NC, NS, L = _info.num_cores, _info.num_subcores, _info.num_lanes
NW = NC * NS

BH = BATCH // 2
NBT = BH // LANES
JT = 8
N_JT = N_E // JT
JH = 16


def _spo_body(occ_hbm, tab_hbm, out_hbm,
              occ_a, occ_b, tab_v, buf_a, buf_b,
              wsem_a, wsem_b, osem_a, osem_b):
    wid = lax.axis_index("s") * NC + lax.axis_index("c")
    d = wid // 2
    h = wid % 2
    b0 = h * BH

    pltpu.sync_copy(tab_hbm.at[d], tab_v)

    def fire_occ(row, occ_v, sem):
        pltpu.async_copy(occ_hbm.at[row, pl.ds(b0, BH)], occ_v, sem)

    def drain_occ(occ_v, sem):
        pltpu.make_async_copy(occ_hbm.at[0, pl.ds(0, BH)], occ_v, sem).wait()

    def produce(occ_v, jhalf, buf):
        @plsc.parallel_loop(0, BH // L, unroll=4)
        def bg_body(g):
            o = occ_v[pl.ds(g * L, L)]
            bt = g // 8
            lo = (g % 8) * L
            for jj in range(JH):
                idx = o + (jhalf * JH + jj) * N_SO
                buf[jj // JT, bt, jj % JT, pl.ds(lo, L)] = (
                    plsc.load_gather(tab_v, [idx]))

    def fire_wb(e, jhalf, buf, sem):
        pltpu.async_copy(
            buf, out_hbm.at[d, e, pl.ds(jhalf * 2, 2), pl.ds(h * NBT, NBT)],
            sem)

    def drain_wb(buf, sem):
        pltpu.make_async_copy(
            buf, out_hbm.at[d, 0, pl.ds(0, 2), pl.ds(0, NBT)], sem).wait()

    fire_occ(0, occ_a, osem_a)
    drain_occ(occ_a, osem_a)
    fire_occ(1, occ_b, osem_b)

    def epair(q, carry):
        e0 = 2 * q
        e1 = e0 + 1

        @pl.when(q > 0)
        def _():
            drain_occ(occ_a, osem_a)

        @pl.when(q > 0)
        def _():
            drain_wb(buf_a, wsem_a)
        produce(occ_a, 0, buf_a)
        fire_wb(e0, 0, buf_a, wsem_a)

        @pl.when(q > 0)
        def _():
            drain_wb(buf_b, wsem_b)
        produce(occ_a, 1, buf_b)
        fire_wb(e0, 1, buf_b, wsem_b)

        @pl.when(q < N_E // 2 - 1)
        def _():
            fire_occ(e0 + 2, occ_a, osem_a)

        drain_occ(occ_b, osem_b)

        drain_wb(buf_a, wsem_a)
        produce(occ_b, 0, buf_a)
        fire_wb(e1, 0, buf_a, wsem_a)

        drain_wb(buf_b, wsem_b)
        produce(occ_b, 1, buf_b)
        fire_wb(e1, 1, buf_b, wsem_b)

        @pl.when(q < N_E // 2 - 1)
        def _():
            fire_occ(e0 + 3, occ_b, osem_b)
        return carry

    lax.fori_loop(0, N_E // 2, epair, 0)
    drain_wb(buf_a, wsem_a)
    drain_wb(buf_b, wsem_b)


@functools.partial(jax.jit, static_argnames=())
def kernel(occ_so, phi_ref):
    occ_t = occ_so.astype(jnp.int32).T
    tab = phi_ref.transpose(0, 2, 1).reshape(N_DET, N_E * N_SO)

    mesh = plsc.VectorSubcoreMesh(core_axis_name="c", subcore_axis_name="s")
    out6 = pl.kernel(
        _spo_body,
        mesh=mesh,
        compiler_params=pltpu.CompilerParams(needs_layout_passes=False),
        out_type=jax.ShapeDtypeStruct(
            (N_DET, N_E, N_JT, BATCH // LANES, JT, LANES), jnp.float32),
        scratch_types=[
            pltpu.VMEM((BH,), jnp.int32),
            pltpu.VMEM((BH,), jnp.int32),
            pltpu.VMEM((N_SO * N_E,), jnp.float32),
            pltpu.VMEM((2, NBT, JT, LANES), jnp.float32),
            pltpu.VMEM((2, NBT, JT, LANES), jnp.float32),
            pltpu.SemaphoreType.DMA,
            pltpu.SemaphoreType.DMA,
            pltpu.SemaphoreType.DMA,
            pltpu.SemaphoreType.DMA,
        ],
    )(occ_t, tab)
    out = out6.transpose(3, 5, 0, 1, 2, 4).reshape(BATCH, N_DET, N_E, N_E)
    return out


# speedup vs baseline: 1.2043x; 1.2043x over previous
import functools

import jax
import jax.numpy as jnp
from jax import lax
from jax.experimental import pallas as pl
from jax.experimental.pallas import tpu as pltpu
from jax.experimental.pallas import tpu_sc as plsc

N_DET = 16
N_SO = 512
N_E = 32
BATCH = 4096
LANES = 128

_info = plsc.get_sparse_core_info()
NC, NS, L = _info.num_cores, _info.num_subcores, _info.num_lanes
NW = NC * NS

BH = BATCH // 2
NBT = BH // LANES
JT = 8
N_JT = N_E // JT
N_STEPS = N_E * N_JT


def _spo_body(occ_hbm, tab_hbm, out_hbm,
              occ_v, tab_v, buf_a, buf_b, wsem_a, wsem_b):
    wid = lax.axis_index("s") * NC + lax.axis_index("c")
    d = wid // 2
    h = wid % 2
    b0 = h * BH

    pltpu.sync_copy(tab_hbm.at[d], tab_v)
    pltpu.sync_copy(occ_hbm.at[:, pl.ds(b0, BH)], occ_v)

    def produce(e, jt, buf):
        jbase = jt * JT

        @plsc.parallel_loop(0, BH // L, unroll=4)
        def bg_body(g):
            o = occ_v[e, pl.ds(g * L, L)]
            bt = g // 8
            lo = (g % 8) * L
            for j in range(JT):
                idx = o + (jbase + j) * N_SO
                buf[bt, j, pl.ds(lo, L)] = plsc.load_gather(tab_v, [idx])

    def fire_wb(e, jt, buf, sem):
        pltpu.async_copy(buf, out_hbm.at[d, e, jt, pl.ds(h * NBT, NBT)], sem)

    def drain_wb(buf, sem):
        pltpu.make_async_copy(buf, out_hbm.at[d, 0, 0, pl.ds(0, NBT)],
                              sem).wait()

    def pair(p, carry):
        ta = 2 * p
        tb = ta + 1
        ea, jta = ta // N_JT, ta % N_JT
        eb, jtb = tb // N_JT, tb % N_JT

        @pl.when(p > 0)
        def _():
            drain_wb(buf_a, wsem_a)
        produce(ea, jta, buf_a)
        fire_wb(ea, jta, buf_a, wsem_a)

        @pl.when(p > 0)
        def _():
            drain_wb(buf_b, wsem_b)
        produce(eb, jtb, buf_b)
        fire_wb(eb, jtb, buf_b, wsem_b)
        return carry

    lax.fori_loop(0, N_STEPS // 2, pair, 0)
    drain_wb(buf_a, wsem_a)
    drain_wb(buf_b, wsem_b)


@functools.partial(jax.jit, static_argnames=())
def kernel(occ_so, phi_ref):
    occ_t = occ_so.astype(jnp.int32).T
    tab = phi_ref.transpose(0, 2, 1).reshape(N_DET, N_E * N_SO)

    mesh = plsc.VectorSubcoreMesh(core_axis_name="c", subcore_axis_name="s")
    out6 = pl.kernel(
        _spo_body,
        mesh=mesh,
        compiler_params=pltpu.CompilerParams(needs_layout_passes=False),
        out_type=jax.ShapeDtypeStruct(
            (N_DET, N_E, N_JT, BATCH // LANES, JT, LANES), jnp.float32),
        scratch_types=[
            pltpu.VMEM((N_E, BH), jnp.int32),
            pltpu.VMEM((N_SO * N_E,), jnp.float32),
            pltpu.VMEM((NBT, JT, LANES), jnp.float32),
            pltpu.VMEM((NBT, JT, LANES), jnp.float32),
            pltpu.SemaphoreType.DMA,
            pltpu.SemaphoreType.DMA,
        ],
    )(occ_t, tab)
    out = out6.transpose(3, 5, 0, 1, 2, 4).reshape(BATCH, N_DET, N_E, N_E)
    return out
